# baseline (speedup 1.0000x reference)
import functools

import jax
import jax.numpy as jnp
from jax import lax
from jax.experimental import pallas as pl
from jax.experimental.pallas import tpu as pltpu
from jax.experimental.pallas import tpu_sc as plsc

_B, _T, _H, _W = 3, 300, 256, 256
_N = 32
_FRAME = _H * _W
_NWORKERS = 32


def _sc_subsample(xf):
    mesh = plsc.VectorSubcoreMesh(core_axis_name="c", subcore_axis_name="s")

    @functools.partial(
        pl.kernel,
        mesh=mesh,
        out_type=jax.ShapeDtypeStruct((_B * _N, _FRAME), jnp.float32),
        scratch_types=[pltpu.SemaphoreType.DMA],
    )
    def k(x_hbm, out_hbm, sem):
        c = lax.axis_index("c")
        s = lax.axis_index("s")
        w = s * 2 + c
        src = lax.div(w * (_T - 1), _N - 1)
        copies = []
        for b in range(_B):
            cp = pltpu.make_async_copy(
                x_hbm.at[b * _T + src], out_hbm.at[b * _N + w], sem
            )
            cp.start()
            copies.append(cp)
        for cp in copies:
            cp.wait()

    return k(xf)


def kernel(x):
    xf = x.reshape(_B * _T, _FRAME)
    out = _sc_subsample(xf)
    return out.reshape(_B, _N, _H, _W)


# speedup vs baseline: 4.8488x; 4.8488x over previous
import functools

import jax
import jax.numpy as jnp
from jax import lax
from jax.experimental import pallas as pl
from jax.experimental.pallas import tpu as pltpu
from jax.experimental.pallas import tpu_sc as plsc

_B, _T, _H, _W = 3, 300, 256, 256
_N = 32
_FRAME = _H * _W
_HALF = _FRAME // 2
_NBUF = 3
_NCHUNK = 2 * _B


def _sc_subsample(xf):
    mesh = plsc.VectorSubcoreMesh(core_axis_name="c", subcore_axis_name="s")

    @functools.partial(
        pl.kernel,
        mesh=mesh,
        out_type=jax.ShapeDtypeStruct((_B * _N * _FRAME,), jnp.float32),
        scratch_types=[
            pltpu.VMEM((_NBUF * _HALF,), jnp.float32),
            pltpu.SemaphoreType.DMA,
            pltpu.SemaphoreType.DMA,
        ],
    )
    def k(x_hbm, out_hbm, buf, sem_in, sem_out):
        c = lax.axis_index("c")
        s = lax.axis_index("s")
        w = s * 2 + c
        src = lax.div(w * (_T - 1), _N - 1)

        def make_in(u):
            b, h = divmod(u, 2)
            off = (b * _T + src) * _FRAME + h * _HALF
            return pltpu.make_async_copy(
                x_hbm.at[pl.ds(pl.multiple_of(off, 8), _HALF)],
                buf.at[pl.ds((u % _NBUF) * _HALF, _HALF)],
                sem_in,
            )

        def make_out(u):
            b, h = divmod(u, 2)
            off = (b * _N + w) * _FRAME + h * _HALF
            return pltpu.make_async_copy(
                buf.at[pl.ds((u % _NBUF) * _HALF, _HALF)],
                out_hbm.at[pl.ds(pl.multiple_of(off, 8), _HALF)],
                sem_out,
            )

        ins = [make_in(u) for u in range(_NCHUNK)]
        outs = [make_out(u) for u in range(_NCHUNK)]
        out_waited = [False] * _NCHUNK

        ins[0].start()
        ins[1].start()
        for t in range(_NCHUNK):
            ins[t].wait()
            outs[t].start()
            u = t + 2
            if u < _NCHUNK:
                if u - _NBUF >= 0:
                    outs[u - _NBUF].wait()
                    out_waited[u - _NBUF] = True
                ins[u].start()
        for t in range(_NCHUNK):
            if not out_waited[t]:
                outs[t].wait()

    return k(xf)


def kernel(x):
    xf = x.reshape(_B * _T * _FRAME)
    out = _sc_subsample(xf)
    return out.reshape(_B, _N, _H, _W)


# speedup vs baseline: 30.6507x; 6.3213x over previous
import functools

import jax
import jax.numpy as jnp
from jax import lax
from jax.experimental import pallas as pl
from jax.experimental.pallas import tpu as pltpu
from jax.experimental.pallas import tpu_sc as plsc

_B, _T, _H, _W = 3, 300, 256, 256
_N = 32
_HROWS = _H // 2
_NBUF = 3
_NCHUNK = 2 * _B


def _sc_subsample(x):
    mesh = plsc.VectorSubcoreMesh(core_axis_name="c", subcore_axis_name="s")

    @functools.partial(
        pl.kernel,
        mesh=mesh,
        out_type=jax.ShapeDtypeStruct((_B, _N, _H, _W), jnp.float32),
        scratch_types=[
            pltpu.VMEM((_NBUF, _HROWS, _W), jnp.float32),
            pltpu.SemaphoreType.DMA,
            pltpu.SemaphoreType.DMA,
        ],
    )
    def k(x_hbm, out_hbm, buf, sem_in, sem_out):
        c = lax.axis_index("c")
        s = lax.axis_index("s")
        w = s * 2 + c
        src = lax.div(w * (_T - 1), _N - 1)

        def make_in(u):
            b, h = divmod(u, 2)
            return pltpu.make_async_copy(
                x_hbm.at[b, src, pl.ds(h * _HROWS, _HROWS)],
                buf.at[u % _NBUF],
                sem_in,
            )

        def make_out(u):
            b, h = divmod(u, 2)
            return pltpu.make_async_copy(
                buf.at[u % _NBUF],
                out_hbm.at[b, w, pl.ds(h * _HROWS, _HROWS)],
                sem_out,
            )

        ins = [make_in(u) for u in range(_NCHUNK)]
        outs = [make_out(u) for u in range(_NCHUNK)]
        out_waited = [False] * _NCHUNK

        ins[0].start()
        ins[1].start()
        for t in range(_NCHUNK):
            ins[t].wait()
            outs[t].start()
            u = t + 2
            if u < _NCHUNK:
                if u - _NBUF >= 0:
                    outs[u - _NBUF].wait()
                    out_waited[u - _NBUF] = True
                ins[u].start()
        for t in range(_NCHUNK):
            if not out_waited[t]:
                outs[t].wait()

    return k(x)


def kernel(x):
    return _sc_subsample(x)
